# baseline (speedup 1.0000x reference)
import functools

import jax
import jax.numpy as jnp
from jax import lax
from jax.experimental import pallas as pl
from jax.experimental.pallas import tpu as pltpu
from jax.experimental.pallas import tpu_sc as plsc

N = 10000
E = 320000
HID = 64

NW = 32
NPW = 320
N_PAD = NW * NPW
EC = 4000
NCH = E // EC
G = 32
NBUF = 4
BUF = 8192
HB = BUF // 2
CAPW = E + HB + G
LB = 2048
GPB = LB // G
NEG = float("-inf")
DEAD = NPW

_MLP_R = 1000

_SC_MESH = dict(core_axis_name="c", subcore_axis_name="s",
                num_cores=2, num_subcores=16)
_SC_PARAMS = dict(needs_layout_passes=False)


def _mlp_body(nin, *refs):
    xrefs = refs[:nin]
    wrefs = refs[nin:2 * nin]
    b1, g, bt, w2, b2, oref = refs[2 * nin:]
    hid = jnp.dot(xrefs[0][...], wrefs[0][...],
                  preferred_element_type=jnp.float32)
    for xr, wr in zip(xrefs[1:], wrefs[1:]):
        hid = hid + jnp.dot(xr[...], wr[...],
                            preferred_element_type=jnp.float32)
    hid = hid + b1[...]
    mu = jnp.mean(hid, axis=1, keepdims=True)
    var = jnp.mean((hid - mu) ** 2, axis=1, keepdims=True)
    hid = (hid - mu) * lax.rsqrt(var + 1e-5) * g[...] + bt[...]
    hid = jnp.maximum(hid, 0.0)
    oref[...] = jnp.dot(hid, w2[...],
                        preferred_element_type=jnp.float32) + b2[...]


def _mlp(xs, W1, b1, g, bt, W2, b2):
    d_out = W2.shape[1]
    nin = len(xs)
    row = lambda i: (i, 0)
    full = lambda i: (0, 0)
    wspec = lambda a: pl.BlockSpec(a.shape, full)
    b1r = b1.reshape(1, HID)
    gr = g.reshape(1, HID)
    btr = bt.reshape(1, HID)
    b2r = b2.reshape(1, d_out)
    w1s = []
    off = 0
    for x in xs:
        w1s.append(W1[off:off + x.shape[1]])
        off += x.shape[1]
    args = tuple(xs) + tuple(w1s) + (b1r, gr, btr, W2, b2r)
    in_specs = (
        [pl.BlockSpec((_MLP_R, x.shape[1]), row) for x in xs]
        + [wspec(w) for w in w1s]
        + [wspec(b1r), wspec(gr), wspec(btr), wspec(W2), wspec(b2r)]
    )
    return pl.pallas_call(
        functools.partial(_mlp_body, nin),
        grid=(N // _MLP_R,),
        in_specs=in_specs,
        out_specs=pl.BlockSpec((_MLP_R, d_out), row),
        out_shape=jax.ShapeDtypeStruct((N, d_out), jnp.float32),
    )(*args)


def _bin_body(src_hbm, dst_hbm, ls_hbm, ld_hbm, cnt_hbm,
              dstb, srcb, rs, rd, cbuf, sem):
    wid = lax.axis_index("s") * 2 + lax.axis_index("c")
    lo = wid * NPW
    hi = lo + NPW
    zeros = jnp.zeros((16,), jnp.int32)

    def flush(fl):
        off = pl.multiple_of(fl & (BUF - 1), 8)
        base = pl.multiple_of(wid * CAPW + fl, 8)
        pltpu.sync_copy(rs.at[pl.ds(off, HB)], ls_hbm.at[pl.ds(base, HB)])
        pltpu.sync_copy(rd.at[pl.ds(off, HB)], ld_hbm.at[pl.ds(base, HB)])

    def chunk_body(c, carry):
        cnt, flushed = carry
        pltpu.sync_copy(dst_hbm.at[pl.ds(c * EC, EC)], dstb)
        pltpu.sync_copy(src_hbm.at[pl.ds(c * EC, EC)], srcb)

        def filt(i, cnt):
            dv = dstb[pl.ds(i * 16, 16)]
            sv = srcb[pl.ds(i * 16, 16)]
            m = (dv >= lo) & (dv < hi)
            pos = plsc.cumsum(jnp.where(m, 1, 0).astype(jnp.int32))
            idx = (cnt + pos - 1) & (BUF - 1)
            plsc.store_scatter(rs, [idx], sv, mask=m)
            plsc.store_scatter(rd, [idx], dv - lo, mask=m)
            return cnt + pos[15]
        cnt = lax.fori_loop(0, EC // 16, filt, cnt)

        @pl.when(cnt - flushed >= HB)
        def _():
            flush(flushed)
        flushed = jnp.where(cnt - flushed >= HB, flushed + HB, flushed)
        return (cnt, flushed)

    cnt, flushed = lax.fori_loop(0, NCH, chunk_body,
                                 (jnp.int32(0), jnp.int32(0)))

    iota = lax.iota(jnp.int32, 16)
    dead = jnp.full((16,), DEAD, jnp.int32)
    for p in range(G // 16):
        idx = (cnt + p * 16 + iota) & (BUF - 1)
        plsc.store_scatter(rs, [idx], zeros)
        plsc.store_scatter(rd, [idx], dead)
    cnt_p = cnt + G

    @pl.when(cnt_p - flushed >= HB)
    def _():
        flush(flushed)
    flushed = jnp.where(cnt_p - flushed >= HB, flushed + HB, flushed)

    @pl.when(cnt_p - flushed > 0)
    def _():
        flush(flushed)

    cbuf[...] = zeros + cnt
    pltpu.sync_copy(cbuf, cnt_hbm.at[pl.ds(pl.multiple_of(wid * 16, 8), 16)])


def _bin_edges(src, dst):
    f = pl.kernel(
        _bin_body,
        out_type=(
            jax.ShapeDtypeStruct((NW * CAPW,), jnp.int32),
            jax.ShapeDtypeStruct((NW * CAPW,), jnp.int32),
            jax.ShapeDtypeStruct((NW * 16,), jnp.int32),
        ),
        mesh=plsc.VectorSubcoreMesh(**_SC_MESH),
        compiler_params=pltpu.CompilerParams(**_SC_PARAMS),
        scratch_types=[
            pltpu.VMEM((EC,), jnp.int32),
            pltpu.VMEM((EC,), jnp.int32),
            pltpu.VMEM((BUF,), jnp.int32),
            pltpu.VMEM((BUF,), jnp.int32),
            pltpu.VMEM((16,), jnp.int32),
            pltpu.SemaphoreType.DMA,
        ],
    )
    return f(src, dst)


def _seg_max_body(d_eff, n_half, h_hbm, ls_hbm, ld_hbm, cnt_hbm, *rest):
    outs = rest[:n_half]
    rest = rest[n_half:]
    cb, lsb, ldb = rest[:3]
    gidxs = rest[3:3 + NBUF]
    stages = rest[3 + NBUF:3 + 2 * NBUF]
    aggr = rest[3 + 2 * NBUF]
    sems = rest[4 + 2 * NBUF:]
    nv = d_eff // 16
    wid = lax.axis_index("s") * 2 + lax.axis_index("c")
    node_lo = wid * NPW
    pltpu.sync_copy(cnt_hbm.at[pl.ds(pl.multiple_of(wid * 16, 8), 16)], cb)
    cnt = cb[...][0]
    ngroups = (cnt + G - 1) // G
    nblocks = (ngroups + GPB - 1) // GPB

    for half in range(n_half):
        out_hbm = outs[half]

        neg16 = jnp.full((16,), NEG, jnp.float32)

        def init_row(r, carry):
            for ci in range(nv):
                aggr[pl.ds(r * d_eff + ci * 16, 16)] = neg16
            return carry
        lax.fori_loop(0, NPW + 1, init_row, 0)

        def blk_body(b, carry):
            lbase = pl.multiple_of(wid * CAPW + b * LB, 8)
            pltpu.sync_copy(ls_hbm.at[pl.ds(lbase, LB)], lsb)
            pltpu.sync_copy(ld_hbm.at[pl.ds(lbase, LB)], ldb)
            ng = jnp.minimum(ngroups - b * GPB, GPB)

            def prep_start(g, s):
                gidx = gidxs[s]
                for p in range(G // 16):
                    v = lsb[pl.ds(g * G + p * 16, 16)]
                    if n_half > 1:
                        v = v * n_half + half
                    gidx[pl.ds(p * 16, 16)] = v
                pltpu.async_copy(h_hbm.at[gidx], stages[s], sems[s])

            def acc(g, stage):
                def jb_body(jb, carry2):
                    dlv = ldb[pl.ds(g * G + jb * 16, 16)] * d_eff
                    for j16 in range(16):
                        abase = dlv[j16]
                        for ci in range(nv):
                            asl = pl.ds(abase + ci * 16, 16)
                            aggr[asl] = jnp.maximum(
                                aggr[asl],
                                stage[jb * 16 + j16, pl.ds(ci * 16, 16)])
                    return carry2
                lax.fori_loop(0, G // 16, jb_body, 0)

            for s in range(NBUF):
                @pl.when(s < ng)
                def _(s=s):
                    prep_start(s, s)

            def round_body(q, carry2):
                for s in range(NBUF):
                    g = q * NBUF + s

                    @pl.when(g < ng)
                    def _(g=g, s=s):
                        pltpu.make_async_copy(h_hbm.at[gidxs[s]],
                                              stages[s], sems[s]).wait()
                        acc(g, stages[s])

                        @pl.when(g + NBUF < ng)
                        def _(g=g, s=s):
                            prep_start(g + NBUF, s)
                return carry2
            lax.fori_loop(0, (ng + NBUF - 1) // NBUF, round_body, 0)
            return carry
        lax.fori_loop(0, nblocks, blk_body, 0)

        def wb_blk(rb, carry):
            def wb_row(rr, carry2):
                abase = (rb * G + rr) * d_eff
                for ci in range(nv):
                    v = aggr[pl.ds(abase + ci * 16, 16)]
                    stages[0][rr, pl.ds(ci * 16, 16)] = jnp.where(
                        v == NEG, 0.0, v)
                return carry2
            lax.fori_loop(0, G, wb_row, 0)
            pltpu.sync_copy(stages[0],
                            out_hbm.at[pl.ds(node_lo + rb * G, G)])
            return carry
        lax.fori_loop(0, NPW // G, wb_blk, 0)


def _seg_max(h, ls, ld, cnts):
    d = h.shape[1]
    n_half = 1 if d <= 256 else 2
    d_eff = d // n_half
    h_in = h.reshape(N * n_half, d_eff)
    out_t = tuple(jax.ShapeDtypeStruct((N_PAD, d_eff), jnp.float32)
                  for _ in range(n_half))
    f = pl.kernel(
        functools.partial(_seg_max_body, d_eff, n_half),
        out_type=out_t if n_half > 1 else out_t[0],
        mesh=plsc.VectorSubcoreMesh(**_SC_MESH),
        compiler_params=pltpu.CompilerParams(**_SC_PARAMS),
        scratch_types=(
            [pltpu.VMEM((16,), jnp.int32),
             pltpu.VMEM((LB,), jnp.int32),
             pltpu.VMEM((LB,), jnp.int32)]
            + [pltpu.VMEM((G,), jnp.int32) for _ in range(NBUF)]
            + [pltpu.VMEM((G, d_eff), jnp.float32) for _ in range(NBUF)]
            + [pltpu.VMEM(((NPW + 1) * d_eff,), jnp.float32)]
            + [pltpu.SemaphoreType.DMA for _ in range(NBUF)]
        ),
    )
    return f(h_in, ls, ld, cnts)


def _final_body(h_ref, lo_ref, hi_ref, o_ref):
    i = pl.program_id(0)
    m = jnp.concatenate(
        [jnp.max(h_ref[...], axis=0, keepdims=True),
         jnp.max(lo_ref[...], axis=0, keepdims=True),
         jnp.max(hi_ref[...], axis=0, keepdims=True)], axis=1)

    @pl.when(i == 0)
    def _():
        o_ref[...] = m

    @pl.when(i > 0)
    def _():
        o_ref[...] = jnp.maximum(o_ref[...], m)

    @pl.when(i == (N // _MLP_R) - 1)
    def _():
        v = o_ref[...]
        o_ref[...] = v * lax.rsqrt(jnp.sum(v * v))


def _finalize(h, a_lo, a_hi):
    d = h.shape[1]
    dh = a_lo.shape[1]
    row = lambda i: (i, 0)
    out = pl.pallas_call(
        _final_body,
        grid=(N // _MLP_R,),
        in_specs=[pl.BlockSpec((_MLP_R, d), row),
                  pl.BlockSpec((_MLP_R, dh), row),
                  pl.BlockSpec((_MLP_R, dh), row)],
        out_specs=pl.BlockSpec((1, d + 2 * dh), lambda i: (0, 0)),
        out_shape=jax.ShapeDtypeStruct((1, d + 2 * dh), jnp.float32),
    )(h, a_lo, a_hi)
    return out.reshape(d + 2 * dh)


def kernel(x, edge_index,
           W1_0, b1_0, g_0, bt_0, W2_0, b2_0,
           W1_1, b1_1, g_1, bt_1, W2_1, b2_1,
           W1_2, b1_2, g_2, bt_2, W2_2, b2_2):
    src = edge_index[0]
    dst = edge_index[1]
    ls, ld, cnts = _bin_edges(src, dst)

    h0 = _mlp([x], W1_0, b1_0, g_0, bt_0, W2_0, b2_0)
    a0 = _seg_max(h0, ls, ld, cnts)
    h1 = _mlp([h0, a0], W1_1, b1_1, g_1, bt_1, W2_1, b2_1)
    a1 = _seg_max(h1, ls, ld, cnts)
    h2 = _mlp([h1, a1], W1_2, b1_2, g_2, bt_2, W2_2, b2_2)
    a2_lo, a2_hi = _seg_max(h2, ls, ld, cnts)
    return _finalize(h2, a2_lo, a2_hi)


# speedup vs baseline: 1.0669x; 1.0669x over previous
import functools

import jax
import jax.numpy as jnp
from jax import lax
from jax.experimental import pallas as pl
from jax.experimental.pallas import tpu as pltpu
from jax.experimental.pallas import tpu_sc as plsc

N = 10000
E = 320000
HID = 64

NW = 32
NPW = 320
N_PAD = NW * NPW
EC = 4000
NCH = E // EC
G = 48
NBUF = 3
WBB = 32
BUF = 8192
HB = BUF // 2
CAPW = E + HB + G
LB = 3072
GPB = LB // G
NEG = float("-inf")
DEAD = NPW

_MLP_R = 1000

_SC_MESH = dict(core_axis_name="c", subcore_axis_name="s",
                num_cores=2, num_subcores=16)
_SC_PARAMS = dict(needs_layout_passes=False)


def _mlp_body(nin, *refs):
    xrefs = refs[:nin]
    wrefs = refs[nin:2 * nin]
    b1, g, bt, w2, b2, oref = refs[2 * nin:]
    hid = jnp.dot(xrefs[0][...], wrefs[0][...],
                  preferred_element_type=jnp.float32)
    for xr, wr in zip(xrefs[1:], wrefs[1:]):
        hid = hid + jnp.dot(xr[...], wr[...],
                            preferred_element_type=jnp.float32)
    hid = hid + b1[...]
    mu = jnp.mean(hid, axis=1, keepdims=True)
    var = jnp.mean((hid - mu) ** 2, axis=1, keepdims=True)
    hid = (hid - mu) * lax.rsqrt(var + 1e-5) * g[...] + bt[...]
    hid = jnp.maximum(hid, 0.0)
    oref[...] = jnp.dot(hid, w2[...],
                        preferred_element_type=jnp.float32) + b2[...]


def _mlp(xs, W1, b1, g, bt, W2, b2):
    d_out = W2.shape[1]
    nin = len(xs)
    row = lambda i: (i, 0)
    full = lambda i: (0, 0)
    wspec = lambda a: pl.BlockSpec(a.shape, full)
    b1r = b1.reshape(1, HID)
    gr = g.reshape(1, HID)
    btr = bt.reshape(1, HID)
    b2r = b2.reshape(1, d_out)
    w1s = []
    off = 0
    for x in xs:
        w1s.append(W1[off:off + x.shape[1]])
        off += x.shape[1]
    args = tuple(xs) + tuple(w1s) + (b1r, gr, btr, W2, b2r)
    in_specs = (
        [pl.BlockSpec((_MLP_R, x.shape[1]), row) for x in xs]
        + [wspec(w) for w in w1s]
        + [wspec(b1r), wspec(gr), wspec(btr), wspec(W2), wspec(b2r)]
    )
    return pl.pallas_call(
        functools.partial(_mlp_body, nin),
        grid=(N // _MLP_R,),
        in_specs=in_specs,
        out_specs=pl.BlockSpec((_MLP_R, d_out), row),
        out_shape=jax.ShapeDtypeStruct((N, d_out), jnp.float32),
    )(*args)


def _bin_body(src_hbm, dst_hbm, ls_hbm, ld_hbm, cnt_hbm,
              dstb, srcb, rs, rd, cbuf, sem):
    wid = lax.axis_index("s") * 2 + lax.axis_index("c")
    lo = wid * NPW
    hi = lo + NPW
    zeros = jnp.zeros((16,), jnp.int32)

    def flush(fl):
        off = pl.multiple_of(fl & (BUF - 1), 8)
        base = pl.multiple_of(wid * CAPW + fl, 8)
        pltpu.sync_copy(rs.at[pl.ds(off, HB)], ls_hbm.at[pl.ds(base, HB)])
        pltpu.sync_copy(rd.at[pl.ds(off, HB)], ld_hbm.at[pl.ds(base, HB)])

    def chunk_body(c, carry):
        cnt, flushed = carry
        pltpu.sync_copy(dst_hbm.at[pl.ds(c * EC, EC)], dstb)
        pltpu.sync_copy(src_hbm.at[pl.ds(c * EC, EC)], srcb)

        def filt(i, cnt):
            dv = dstb[pl.ds(i * 16, 16)]
            sv = srcb[pl.ds(i * 16, 16)]
            m = (dv >= lo) & (dv < hi)
            pos = plsc.cumsum(jnp.where(m, 1, 0).astype(jnp.int32))
            idx = (cnt + pos - 1) & (BUF - 1)
            plsc.store_scatter(rs, [idx], sv, mask=m)
            plsc.store_scatter(rd, [idx], dv - lo, mask=m)
            return cnt + pos[15]
        cnt = lax.fori_loop(0, EC // 16, filt, cnt)

        @pl.when(cnt - flushed >= HB)
        def _():
            flush(flushed)
        flushed = jnp.where(cnt - flushed >= HB, flushed + HB, flushed)
        return (cnt, flushed)

    cnt, flushed = lax.fori_loop(0, NCH, chunk_body,
                                 (jnp.int32(0), jnp.int32(0)))

    iota = lax.iota(jnp.int32, 16)
    dead = jnp.full((16,), DEAD, jnp.int32)
    for p in range(G // 16):
        idx = (cnt + p * 16 + iota) & (BUF - 1)
        plsc.store_scatter(rs, [idx], zeros)
        plsc.store_scatter(rd, [idx], dead)
    cnt_p = cnt + G

    @pl.when(cnt_p - flushed >= HB)
    def _():
        flush(flushed)
    flushed = jnp.where(cnt_p - flushed >= HB, flushed + HB, flushed)

    @pl.when(cnt_p - flushed > 0)
    def _():
        flush(flushed)

    cbuf[...] = zeros + cnt
    pltpu.sync_copy(cbuf, cnt_hbm.at[pl.ds(pl.multiple_of(wid * 16, 8), 16)])


def _bin_edges(src, dst):
    f = pl.kernel(
        _bin_body,
        out_type=(
            jax.ShapeDtypeStruct((NW * CAPW,), jnp.int32),
            jax.ShapeDtypeStruct((NW * CAPW,), jnp.int32),
            jax.ShapeDtypeStruct((NW * 16,), jnp.int32),
        ),
        mesh=plsc.VectorSubcoreMesh(**_SC_MESH),
        compiler_params=pltpu.CompilerParams(**_SC_PARAMS),
        scratch_types=[
            pltpu.VMEM((EC,), jnp.int32),
            pltpu.VMEM((EC,), jnp.int32),
            pltpu.VMEM((BUF,), jnp.int32),
            pltpu.VMEM((BUF,), jnp.int32),
            pltpu.VMEM((16,), jnp.int32),
            pltpu.SemaphoreType.DMA,
        ],
    )
    return f(src, dst)


def _seg_max_body(d_eff, n_half, h_hbm, ls_hbm, ld_hbm, cnt_hbm, *rest):
    outs = rest[:n_half]
    rest = rest[n_half:]
    cb, lsb, ldb = rest[:3]
    gidxs = rest[3:3 + NBUF]
    stages = rest[3 + NBUF:3 + 2 * NBUF]
    aggr = rest[3 + 2 * NBUF]
    sems = rest[4 + 2 * NBUF:]
    nv = d_eff // 16
    wid = lax.axis_index("s") * 2 + lax.axis_index("c")
    node_lo = wid * NPW
    pltpu.sync_copy(cnt_hbm.at[pl.ds(pl.multiple_of(wid * 16, 8), 16)], cb)
    cnt = cb[...][0]
    ngroups = (cnt + G - 1) // G
    nblocks = (ngroups + GPB - 1) // GPB

    for half in range(n_half):
        out_hbm = outs[half]

        neg16 = jnp.full((16,), NEG, jnp.float32)

        def init_row(r, carry):
            for ci in range(nv):
                aggr[pl.ds(r * d_eff + ci * 16, 16)] = neg16
            return carry
        lax.fori_loop(0, NPW + 1, init_row, 0)

        def blk_body(b, carry):
            lbase = pl.multiple_of(wid * CAPW + b * LB, 8)
            pltpu.sync_copy(ls_hbm.at[pl.ds(lbase, LB)], lsb)
            pltpu.sync_copy(ld_hbm.at[pl.ds(lbase, LB)], ldb)
            ng = jnp.minimum(ngroups - b * GPB, GPB)

            def prep_start(g, s):
                gidx = gidxs[s]
                for p in range(G // 16):
                    v = lsb[pl.ds(g * G + p * 16, 16)]
                    if n_half > 1:
                        v = v * n_half + half
                    gidx[pl.ds(p * 16, 16)] = v
                pltpu.async_copy(h_hbm.at[gidx], stages[s], sems[s])

            def acc(g, stage):
                def jb_body(jb, carry2):
                    dlv = ldb[pl.ds(g * G + jb * 16, 16)] * d_eff
                    for j16 in range(16):
                        abase = dlv[j16]
                        for ci in range(nv):
                            asl = pl.ds(abase + ci * 16, 16)
                            aggr[asl] = jnp.maximum(
                                aggr[asl],
                                stage[jb * 16 + j16, pl.ds(ci * 16, 16)])
                    return carry2
                lax.fori_loop(0, G // 16, jb_body, 0)

            for s in range(NBUF):
                @pl.when(s < ng)
                def _(s=s):
                    prep_start(s, s)

            def round_body(q, carry2):
                for s in range(NBUF):
                    g = q * NBUF + s

                    @pl.when(g < ng)
                    def _(g=g, s=s):
                        pltpu.make_async_copy(h_hbm.at[gidxs[s]],
                                              stages[s], sems[s]).wait()
                        acc(g, stages[s])

                        @pl.when(g + NBUF < ng)
                        def _(g=g, s=s):
                            prep_start(g + NBUF, s)
                return carry2
            lax.fori_loop(0, (ng + NBUF - 1) // NBUF, round_body, 0)
            return carry
        lax.fori_loop(0, nblocks, blk_body, 0)

        def wb_blk(rb, carry):
            def wb_row(rr, carry2):
                abase = (rb * WBB + rr) * d_eff
                for ci in range(nv):
                    v = aggr[pl.ds(abase + ci * 16, 16)]
                    stages[0][rr, pl.ds(ci * 16, 16)] = jnp.where(
                        v == NEG, 0.0, v)
                return carry2
            lax.fori_loop(0, WBB, wb_row, 0)
            pltpu.sync_copy(stages[0].at[pl.ds(0, WBB)],
                            out_hbm.at[pl.ds(node_lo + rb * WBB, WBB)])
            return carry
        lax.fori_loop(0, NPW // WBB, wb_blk, 0)


def _seg_max(h, ls, ld, cnts):
    d = h.shape[1]
    n_half = 1 if d <= 256 else 2
    d_eff = d // n_half
    h_in = h.reshape(N * n_half, d_eff)
    out_t = tuple(jax.ShapeDtypeStruct((N_PAD, d_eff), jnp.float32)
                  for _ in range(n_half))
    f = pl.kernel(
        functools.partial(_seg_max_body, d_eff, n_half),
        out_type=out_t if n_half > 1 else out_t[0],
        mesh=plsc.VectorSubcoreMesh(**_SC_MESH),
        compiler_params=pltpu.CompilerParams(**_SC_PARAMS),
        scratch_types=(
            [pltpu.VMEM((16,), jnp.int32),
             pltpu.VMEM((LB,), jnp.int32),
             pltpu.VMEM((LB,), jnp.int32)]
            + [pltpu.VMEM((G,), jnp.int32) for _ in range(NBUF)]
            + [pltpu.VMEM((G, d_eff), jnp.float32) for _ in range(NBUF)]
            + [pltpu.VMEM(((NPW + 1) * d_eff,), jnp.float32)]
            + [pltpu.SemaphoreType.DMA for _ in range(NBUF)]
        ),
    )
    return f(h_in, ls, ld, cnts)


def _final_body(h_ref, lo_ref, hi_ref, o_ref):
    i = pl.program_id(0)
    m = jnp.concatenate(
        [jnp.max(h_ref[...], axis=0, keepdims=True),
         jnp.max(lo_ref[...], axis=0, keepdims=True),
         jnp.max(hi_ref[...], axis=0, keepdims=True)], axis=1)

    @pl.when(i == 0)
    def _():
        o_ref[...] = m

    @pl.when(i > 0)
    def _():
        o_ref[...] = jnp.maximum(o_ref[...], m)

    @pl.when(i == (N // _MLP_R) - 1)
    def _():
        v = o_ref[...]
        o_ref[...] = v * lax.rsqrt(jnp.sum(v * v))


def _finalize(h, a_lo, a_hi):
    d = h.shape[1]
    dh = a_lo.shape[1]
    row = lambda i: (i, 0)
    out = pl.pallas_call(
        _final_body,
        grid=(N // _MLP_R,),
        in_specs=[pl.BlockSpec((_MLP_R, d), row),
                  pl.BlockSpec((_MLP_R, dh), row),
                  pl.BlockSpec((_MLP_R, dh), row)],
        out_specs=pl.BlockSpec((1, d + 2 * dh), lambda i: (0, 0)),
        out_shape=jax.ShapeDtypeStruct((1, d + 2 * dh), jnp.float32),
    )(h, a_lo, a_hi)
    return out.reshape(d + 2 * dh)


def kernel(x, edge_index,
           W1_0, b1_0, g_0, bt_0, W2_0, b2_0,
           W1_1, b1_1, g_1, bt_1, W2_1, b2_1,
           W1_2, b1_2, g_2, bt_2, W2_2, b2_2):
    src = edge_index[0]
    dst = edge_index[1]
    ls, ld, cnts = _bin_edges(src, dst)

    h0 = _mlp([x], W1_0, b1_0, g_0, bt_0, W2_0, b2_0)
    a0 = _seg_max(h0, ls, ld, cnts)
    h1 = _mlp([h0, a0], W1_1, b1_1, g_1, bt_1, W2_1, b2_1)
    a1 = _seg_max(h1, ls, ld, cnts)
    h2 = _mlp([h1, a1], W1_2, b1_2, g_2, bt_2, W2_2, b2_2)
    a2_lo, a2_hi = _seg_max(h2, ls, ld, cnts)
    return _finalize(h2, a2_lo, a2_hi)
